# Initial kernel scaffold; baseline (speedup 1.0000x reference)
import jax
import jax.numpy as jnp
from jax.experimental import pallas as pl


def kernel(embeddings, labels):
    raise NotImplementedError("write your pallas kernel here")


# speedup vs baseline: 1.2530x; 1.2530x over previous
import jax
import jax.numpy as jnp
from jax.experimental import pallas as pl
from jax.experimental.pallas import tpu as pltpu

BATCH = 4096
DIM = 128
BLK = 512
NBLK = BATCH // BLK


def _loss_body(a_ref, b_ref, lab_i_ref, lab_j_ref, out_ref, acc_ref):
    i = pl.program_id(0)

    @pl.when(i == 0)
    def _init():
        acc_ref[0] = 0.0
        acc_ref[1] = 0.0
        acc_ref[2] = 0.0
        acc_ref[3] = 0.0

    a = a_ref[...]
    b = b_ref[...]
    g = jax.lax.dot_general(
        a, b, dimension_numbers=(((1,), (1,)), ((), ())),
        preferred_element_type=jnp.float32)
    an = jnp.sum(a * a, axis=1, keepdims=True)
    bn = jnp.sum(b * b, axis=1)[None, :]
    sq = an - 2.0 * g + bn
    dist = jnp.sqrt(jnp.maximum(sq, 1e-16))

    li = lab_i_ref[...]
    lj = lab_j_ref[...]
    pos_mask = li == lj

    zero = jnp.zeros_like(dist)
    pos_sum = jnp.sum(jnp.where(pos_mask, dist, zero))
    pos_cnt = jnp.sum(pos_mask.astype(jnp.float32))
    neg_l = jnp.maximum(1.0 - dist, 0.0)
    neg_sum = jnp.sum(jnp.where(pos_mask, zero, neg_l))
    neg_cnt = jnp.sum(jnp.where(pos_mask | (dist >= 1.0), zero,
                                jnp.ones_like(dist)))

    acc_ref[0] = acc_ref[0] + pos_sum
    acc_ref[1] = acc_ref[1] + pos_cnt
    acc_ref[2] = acc_ref[2] + neg_sum
    acc_ref[3] = acc_ref[3] + neg_cnt

    @pl.when(i == NBLK - 1)
    def _fini():
        pos_avg = acc_ref[0] / jnp.maximum(acc_ref[1], 1.0)
        neg_avg = acc_ref[2] / jnp.maximum(acc_ref[3], 1.0)
        out_ref[...] = jnp.reshape(pos_avg + neg_avg, (1, 1))


def kernel(embeddings, labels):
    emb = embeddings.astype(jnp.float32)
    lab_col = labels.astype(jnp.int32).reshape(BATCH, 1)
    lab_row = labels.astype(jnp.int32).reshape(1, BATCH)
    out = pl.pallas_call(
        _loss_body,
        grid=(NBLK,),
        in_specs=[
            pl.BlockSpec((BLK, DIM), lambda i: (i, 0)),
            pl.BlockSpec((BATCH, DIM), lambda i: (0, 0)),
            pl.BlockSpec((BLK, 1), lambda i: (i, 0)),
            pl.BlockSpec((1, BATCH), lambda i: (0, 0)),
        ],
        out_specs=pl.BlockSpec((1, 1), lambda i: (0, 0)),
        out_shape=jax.ShapeDtypeStruct((1, 1), jnp.float32),
        scratch_shapes=[pltpu.SMEM((4,), jnp.float32)],
    )(emb, emb, lab_col, lab_row)
    return out[0, 0]


# speedup vs baseline: 1.4692x; 1.1725x over previous
import jax
import jax.numpy as jnp
from jax.experimental import pallas as pl
from jax.experimental.pallas import tpu as pltpu

BATCH = 4096
DIM = 128
BLK = 512
NBLK = BATCH // BLK


def _loss_body(a_ref, b_ref, lab_i_ref, lab_j_ref, out_ref, acc_ref):
    i = pl.program_id(0)
    j = pl.program_id(1)

    @pl.when((i == 0) & (j == 0))
    def _init():
        acc_ref[0] = 0.0
        acc_ref[1] = 0.0
        acc_ref[2] = 0.0
        acc_ref[3] = 0.0

    @pl.when(j >= i)
    def _compute():
        a = a_ref[...]
        b = b_ref[...]
        g = jax.lax.dot_general(
            a, b, dimension_numbers=(((1,), (1,)), ((), ())),
            preferred_element_type=jnp.float32)
        an = jnp.sum(a * a, axis=1, keepdims=True)
        bn = jnp.sum(b * b, axis=1)[None, :]
        sq = an - 2.0 * g + bn
        dist = jnp.sqrt(jnp.maximum(sq, 1e-16))

        pos_mask = lab_i_ref[...] == lab_j_ref[...]
        zero = jnp.zeros_like(dist)
        pos_sum = jnp.sum(jnp.where(pos_mask, dist, zero))
        pos_cnt = jnp.sum(pos_mask.astype(jnp.float32))
        neg_l = jnp.maximum(1.0 - dist, 0.0)
        neg_sum = jnp.sum(jnp.where(pos_mask, zero, neg_l))
        neg_cnt = jnp.sum(jnp.where(pos_mask | (dist >= 1.0), zero,
                                    jnp.ones_like(dist)))

        w = jnp.where(i == j, 1.0, 2.0)
        acc_ref[0] = acc_ref[0] + w * pos_sum
        acc_ref[1] = acc_ref[1] + w * pos_cnt
        acc_ref[2] = acc_ref[2] + w * neg_sum
        acc_ref[3] = acc_ref[3] + w * neg_cnt

    @pl.when((i == NBLK - 1) & (j == NBLK - 1))
    def _fini():
        pos_avg = acc_ref[0] / jnp.maximum(acc_ref[1], 1.0)
        neg_avg = acc_ref[2] / jnp.maximum(acc_ref[3], 1.0)
        out_ref[...] = jnp.reshape(pos_avg + neg_avg, (1, 1))


def kernel(embeddings, labels):
    emb = embeddings.astype(jnp.float32)
    lab_col = labels.astype(jnp.int32).reshape(BATCH, 1)
    lab_row = labels.astype(jnp.int32).reshape(1, BATCH)
    out = pl.pallas_call(
        _loss_body,
        grid=(NBLK, NBLK),
        in_specs=[
            pl.BlockSpec((BLK, DIM), lambda i, j: (i, 0)),
            pl.BlockSpec((BLK, DIM), lambda i, j: (j, 0)),
            pl.BlockSpec((BLK, 1), lambda i, j: (i, 0)),
            pl.BlockSpec((1, BLK), lambda i, j: (0, j)),
        ],
        out_specs=pl.BlockSpec((1, 1), lambda i, j: (0, 0)),
        out_shape=jax.ShapeDtypeStruct((1, 1), jnp.float32),
        scratch_shapes=[pltpu.SMEM((4,), jnp.float32)],
    )(emb, emb, lab_col, lab_row)
    return out[0, 0]
